# baseline (speedup 1.0000x reference)
import functools

import jax
import jax.numpy as jnp
from jax import lax
from jax.experimental import pallas as pl
from jax.experimental.pallas import tpu as pltpu
from jax.experimental.pallas import tpu_sc as plsc

VOCAB = 1000000
EMBED = 64
B = 1024
L = 200
N = B * L

_INFO = plsc.get_sparse_core_info()
NC = _INFO.num_cores
NS = _INFO.num_subcores
NW = NC * NS
PER_W = N // NW
SUB = 128
CH = 640
NG = CH // SUB
NCH = PER_W // CH
IROWS = PER_W // SUB

TC_GRID = 32
RPB = B // TC_GRID
EPB = RPB * L


def _gather_kernel(ids_hbm, table_hbm, out_hbm, idx_v, rows0, rows1,
                   sg0, sg1, so0, so1):
    wid = lax.axis_index("s") * NC + lax.axis_index("c")
    base = wid * PER_W
    rows = (rows0, rows1)
    sg = (sg0, sg1)
    so = (so0, so1)

    pltpu.sync_copy(ids_hbm.at[pl.ds(wid * IROWS, IROWS)], idx_v)

    def gather_cp(b, c, g):
        return pltpu.make_async_copy(
            table_hbm.at[idx_v.at[c * NG + g]],
            rows[b].at[pl.ds(g * SUB, SUB)], sg[b])

    def fire(b, c):
        for g in range(NG):
            gather_cp(b, c, g).start()

    def wait_gather(b, c):
        for g in range(NG):
            gather_cp(b, c, g).wait()

    def out_cp(b, c):
        return pltpu.make_async_copy(
            rows[b],
            out_hbm.at[pl.ds(base + c * CH, CH), pl.ds(0, EMBED)], so[b])

    fire(0, 0)
    for c in range(NCH):
        b = c & 1
        if c + 1 < NCH:
            if c >= 1:
                out_cp(1 - b, c - 1).wait()
            fire(1 - b, c + 1)
        wait_gather(b, c)
        out_cp(b, c).start()
    out_cp(NCH & 1, NCH - 2).wait()
    out_cp(1 - (NCH & 1), NCH - 1).wait()


def _ln_kernel(g_ref, gamma_ref, beta_ref, out_ref):
    x = g_ref[...][:, :EMBED]
    mean = jnp.mean(x, axis=-1, keepdims=True)
    d = x - mean
    var = jnp.mean(d * d, axis=-1, keepdims=True)
    normed = d * lax.rsqrt(var + 1e-12)
    y = normed * gamma_ref[0] + beta_ref[0]
    out_ref[...] = y.reshape(RPB, L, EMBED)


@jax.jit
def _run(ids2d, table, gamma, beta):
    ids128 = jnp.minimum(ids2d, VOCAB - 1).reshape(N // SUB, SUB)
    mesh = plsc.VectorSubcoreMesh(core_axis_name="c", subcore_axis_name="s")
    gathered = pl.kernel(
        _gather_kernel,
        mesh=mesh,
        out_type=jax.ShapeDtypeStruct((N, 2 * EMBED), jnp.float32),
        scratch_types=[
            pltpu.VMEM((IROWS, SUB), jnp.int32),
            pltpu.VMEM((CH, EMBED), jnp.float32),
            pltpu.VMEM((CH, EMBED), jnp.float32),
            pltpu.SemaphoreType.DMA,
            pltpu.SemaphoreType.DMA,
            pltpu.SemaphoreType.DMA,
            pltpu.SemaphoreType.DMA,
        ],
        compiler_params=pltpu.CompilerParams(
            needs_layout_passes=False, use_tc_tiling_on_sc=False,
            skip_device_barrier=True),
    )(ids128, table)

    out = pl.pallas_call(
        _ln_kernel,
        grid=(TC_GRID,),
        in_specs=[
            pl.BlockSpec((EPB, 2 * EMBED), lambda b: (b, 0)),
            pl.BlockSpec((1, EMBED), lambda b: (0, 0)),
            pl.BlockSpec((1, EMBED), lambda b: (0, 0)),
        ],
        out_specs=pl.BlockSpec((RPB, L, EMBED), lambda b: (b, 0, 0)),
        out_shape=jax.ShapeDtypeStruct((B, L, EMBED), jnp.float32),
        compiler_params=pltpu.CompilerParams(
            dimension_semantics=("arbitrary",)),
    )(gathered, gamma.reshape(1, EMBED), beta.reshape(1, EMBED))
    return out


def kernel(input_ids, word_table, ln_gamma, ln_beta):
    return _run(input_ids.astype(jnp.int32), word_table, ln_gamma, ln_beta)
